# baseline (speedup 1.0000x reference)
import jax
import jax.numpy as jnp
from jax import lax
from jax.experimental import pallas as pl
from jax.experimental.pallas import tpu as pltpu
from jax.experimental.pallas import tpu_sc as plsc

_INFO = plsc.get_sparse_core_info()
_NC = _INFO.num_cores
_NS = _INFO.num_subcores
_NW = _NC * _NS
_L = _INFO.num_lanes

_B = 16384
_D = 64
_V = 1_000_000
_BPW = _B // _NW
_NGRP = _BPW // _L
_CH = 128
_EPW = _D * _BPW
_NCH = _EPW // _CH


def _body(uidx_h, iidx_h, utab_h, itab_h, w_h, b_h, out_h,
          idx_v, u_cols, i_cols, w_v, b_v, out_v, sem):
    wid = lax.axis_index("s") * _NC + lax.axis_index("c")

    def gather(idx_hbm, tab_hbm, cols):
        pltpu.sync_copy(idx_hbm.at[wid], idx_v)

        def fire(k, carry):
            sl = pl.ds(k * _CH, _CH)
            pltpu.async_copy(tab_hbm.at[idx_v.at[sl]], cols.at[sl], sem)
            return carry

        lax.fori_loop(0, _NCH, fire, 0)
        pltpu.make_async_copy(tab_hbm.at[pl.ds(0, _EPW)], cols, sem).wait()

    gather(uidx_h, utab_h, u_cols)
    gather(iidx_h, itab_h, i_cols)

    pltpu.sync_copy(w_h, w_v)
    pltpu.sync_copy(b_h, b_v)
    bvec = b_v[:]

    def over_d(d, accs):
        wd = w_v[d, pl.ds(0, _L)]
        base = d * _BPW
        new = []
        for g in range(_NGRP):
            u = u_cols[pl.ds(base + g * _L, _L)]
            v = i_cols[pl.ds(base + g * _L, _L)]
            new.append(accs[g] + u * v * wd)
        return tuple(new)

    accs = lax.fori_loop(0, _D, over_d,
                         tuple(jnp.zeros((_L,), jnp.float32)
                               for _ in range(_NGRP)))
    for g in range(_NGRP):
        logits = accs[g] + bvec
        out_v[pl.ds(g * _L, _L)] = 1.0 / (1.0 + jnp.exp(-logits))

    pltpu.sync_copy(out_v, out_h.at[wid])


@jax.jit
def _gmf(uidx, iidx, utab_t, itab_t, w, b):
    mesh = plsc.VectorSubcoreMesh(core_axis_name="c", subcore_axis_name="s")
    return pl.kernel(
        _body,
        out_type=jax.ShapeDtypeStruct((_NW, _BPW), jnp.float32),
        mesh=mesh,
        compiler_params=pltpu.CompilerParams(
            needs_layout_passes=False, use_tc_tiling_on_sc=False),
        scratch_types=[
            pltpu.VMEM((_EPW,), jnp.int32),
            pltpu.VMEM((_EPW,), jnp.float32),
            pltpu.VMEM((_EPW,), jnp.float32),
            pltpu.VMEM((_D, _L), jnp.float32),
            pltpu.VMEM((_L,), jnp.float32),
            pltpu.VMEM((_BPW,), jnp.float32),
            pltpu.SemaphoreType.DMA,
        ],
    )(uidx, iidx, utab_t, itab_t, w, b)


def kernel(user_indices, item_indices, user_table, item_table, affine_w, affine_b):
    d_off = jnp.arange(_D, dtype=jnp.int32) * _V
    uidx = (user_indices.astype(jnp.int32).reshape(_NW, 1, _BPW)
            + d_off.reshape(1, _D, 1)).reshape(_NW, _EPW)
    iidx = (item_indices.astype(jnp.int32).reshape(_NW, 1, _BPW)
            + d_off.reshape(1, _D, 1)).reshape(_NW, _EPW)
    w = jnp.broadcast_to(affine_w.reshape(_D, 1), (_D, _L))
    b = jnp.broadcast_to(affine_b.reshape(1), (_L,))
    out = _gmf(uidx, iidx,
               user_table.T.reshape(-1), item_table.T.reshape(-1), w, b)
    return out.reshape(_B, 1)


# speedup vs baseline: 9.0701x; 9.0701x over previous
import jax
import jax.numpy as jnp
from jax import lax
from jax.experimental import pallas as pl
from jax.experimental.pallas import tpu as pltpu
from jax.experimental.pallas import tpu_sc as plsc

_INFO = plsc.get_sparse_core_info()
_NC = _INFO.num_cores
_NS = _INFO.num_subcores
_NW = _NC * _NS
_L = _INFO.num_lanes

_B = 16384
_D = 64
_DC = _D // _L
_BPW = _B // _NW
_HALF = _BPW // 2
_CH = 128
_NCH = _HALF // _CH
_NGRP = _HALF // _L


def _body(upix_h, ipix_h, uh_h, ih_h, utab_h, itab_h, w_h, b_h, out_h,
          pix_v, hu_v, hi_v, u_rows, i_rows, w_v, b_v, out_v, sem):
    wid = lax.axis_index("s") * _NC + lax.axis_index("c")

    pltpu.sync_copy(w_h, w_v)
    pltpu.sync_copy(b_h, b_v)
    wc = [w_v[c] for c in range(_DC)]
    bvec = b_v[:]

    for half in range(2):
        pltpu.sync_copy(upix_h.at[wid, half], pix_v.at[0])
        pltpu.sync_copy(ipix_h.at[wid, half], pix_v.at[1])
        pltpu.sync_copy(uh_h.at[wid, half], hu_v)
        pltpu.sync_copy(ih_h.at[wid, half], hi_v)
        copies = []
        for j in range(_NCH):
            dst = pl.ds(j * _CH, _CH)
            copies.append(pltpu.async_copy(
                utab_h.at[pix_v.at[0, j]], u_rows.at[dst], sem))
            copies.append(pltpu.async_copy(
                itab_h.at[pix_v.at[1, j]], i_rows.at[dst], sem))
        for c in copies:
            c.wait()

        def group(g, carry):
            base = g * _L
            huv = hu_v[0, pl.ds(base, _L)]
            hiv = hi_v[0, pl.ds(base, _L)]
            acc = bvec
            iota = lax.iota(jnp.int32, _L)
            for r in range(_L):
                row = base + r
                hu = huv[r] * _D
                hi = hiv[r] * _D
                s = None
                for c in range(_DC):
                    u = u_rows[row, pl.ds(hu + c * _L, _L)]
                    v = i_rows[row, pl.ds(hi + c * _L, _L)]
                    t = u * v * wc[c]
                    s = t if s is None else s + t
                tot = jnp.sum(s)
                acc = acc + jnp.where(iota == r, tot,
                                      jnp.zeros((_L,), jnp.float32))
            rating = 1.0 / (1.0 + jnp.exp(-acc))
            out_v[pl.ds(half * _HALF + base, _L)] = rating
            return carry

        lax.fori_loop(0, _NGRP, group, 0)

    pltpu.sync_copy(out_v, out_h.at[wid])


@jax.jit
def _gmf(upix, ipix, uh, ih, utab2, itab2, w, b):
    mesh = plsc.VectorSubcoreMesh(core_axis_name="c", subcore_axis_name="s")
    return pl.kernel(
        _body,
        out_type=jax.ShapeDtypeStruct((_NW, _BPW), jnp.float32),
        mesh=mesh,
        compiler_params=pltpu.CompilerParams(
            needs_layout_passes=False, use_tc_tiling_on_sc=True),
        scratch_types=[
            pltpu.VMEM((2, _NCH, _CH), jnp.int32),
            pltpu.VMEM((1, _HALF), jnp.int32),
            pltpu.VMEM((1, _HALF), jnp.int32),
            pltpu.VMEM((_HALF, 2 * _D), jnp.float32),
            pltpu.VMEM((_HALF, 2 * _D), jnp.float32),
            pltpu.VMEM((_DC, _L), jnp.float32),
            pltpu.VMEM((_L,), jnp.float32),
            pltpu.VMEM((_BPW,), jnp.float32),
            pltpu.SemaphoreType.DMA,
        ],
    )(upix, ipix, uh, ih, utab2, itab2, w, b)


def kernel(user_indices, item_indices, user_table, item_table, affine_w, affine_b):
    ui = user_indices.astype(jnp.int32)
    ii = item_indices.astype(jnp.int32)
    upix = (ui >> 1).reshape(_NW, 2, _NCH, _CH)
    ipix = (ii >> 1).reshape(_NW, 2, _NCH, _CH)
    uh = (ui & 1).reshape(_NW, 2, 1, _HALF)
    ih = (ii & 1).reshape(_NW, 2, 1, _HALF)
    w = affine_w.reshape(_DC, _L)
    b = jnp.broadcast_to(affine_b.reshape(1), (_L,))
    out = _gmf(upix, ipix, uh, ih,
               user_table.reshape(500000, 128),
               item_table.reshape(500000, 128), w, b)
    return out.reshape(_B, 1)
